# baseline (speedup 1.0000x reference)
import functools

import jax
import jax.numpy as jnp
from jax import lax
from jax.experimental import pallas as pl
from jax.experimental.pallas import tpu as pltpu
from jax.experimental.pallas import tpu_sc as plsc

_SLEN = 256
_DIM = 512
_PDIM = 64
_REV_W = 512

_NC = 2
_NS = 16
_TPW = 8


def _rev_body(pe_k_ref, pe_v_ref, rk_ref, rv_ref):
    u = lax.broadcasted_iota(jnp.int32, (_REV_W, _SLEN + 1), 0)
    r = lax.broadcasted_iota(jnp.int32, (_REV_W, _SLEN + 1), 1)
    sel = (r == jnp.maximum(_SLEN - u, 0)).astype(jnp.float32)
    dn = (((0,), (1,)), ((), ()))
    rk_ref[...] = lax.dot_general(pe_k_ref[pl.ds(0, _SLEN + 1)], sel, dn,
                                  preferred_element_type=jnp.float32)
    rv_ref[...] = lax.dot_general(pe_v_ref[pl.ds(0, _SLEN + 1)], sel, dn,
                                  preferred_element_type=jnp.float32)


def _tc_rev(pe_k, pe_v):
    return pl.pallas_call(
        _rev_body,
        out_shape=(
            jax.ShapeDtypeStruct((_PDIM, _REV_W), jnp.float32),
            jax.ShapeDtypeStruct((_PDIM, _REV_W), jnp.float32),
        ),
        in_specs=[
            pl.BlockSpec(memory_space=pltpu.VMEM),
            pl.BlockSpec(memory_space=pltpu.VMEM),
        ],
        out_specs=(
            pl.BlockSpec(memory_space=pltpu.VMEM),
            pl.BlockSpec(memory_space=pltpu.VMEM),
        ),
    )(pe_k, pe_v)


def _out_body(x_ref, comp_ref, bases_ref, root_ref, bias_ref, o_ref):
    x = x_ref[...]
    w0 = root_ref[...]
    w1 = jnp.zeros((_DIM, _DIM), jnp.float32)
    for b in range(4):
        w0 = w0 + comp_ref[0, b] * bases_ref[b]
        w1 = w1 + comp_ref[1, b] * bases_ref[b]
    row = lax.broadcasted_iota(jnp.int32, (_SLEN, _SLEN), 0)
    col = lax.broadcasted_iota(jnp.int32, (_SLEN, _SLEN), 1)
    tri = (col < row).astype(jnp.float32)
    s = jnp.dot(tri, x, preferred_element_type=jnp.float32)
    n = lax.broadcasted_iota(jnp.int32, (_SLEN, 1), 0).astype(jnp.float32)
    p = s / jnp.maximum(n, 1.0)
    o_ref[...] = (
        jnp.dot(x, w0, preferred_element_type=jnp.float32)
        + jnp.dot(p, w1, preferred_element_type=jnp.float32)
        + bias_ref[...]
    )


def _tc_out(x, comp, bases, root, bias):
    return pl.pallas_call(
        _out_body,
        out_shape=jax.ShapeDtypeStruct((_SLEN, _DIM), jnp.float32),
        in_specs=[
            pl.BlockSpec(memory_space=pltpu.VMEM),
            pl.BlockSpec(memory_space=pltpu.SMEM),
            pl.BlockSpec(memory_space=pltpu.VMEM),
            pl.BlockSpec(memory_space=pltpu.VMEM),
            pl.BlockSpec(memory_space=pltpu.VMEM),
        ],
        out_specs=pl.BlockSpec(memory_space=pltpu.VMEM),
    )(x, comp, bases, root, bias.reshape(1, _DIM))


def _rel_body(rk_hbm, outk_hbm, ph, bb0, bb1, sem):
    wid = lax.axis_index("s") * _NC + lax.axis_index("c")
    t0 = wid * _TPW
    bufs = (bb0, bb1)

    hs = [pltpu.async_copy(rk_hbm.at[d], ph.at[pl.ds(d * _REV_W, _REV_W)],
                           sem) for d in range(_PDIM)]
    for h in hs:
        h.wait()

    lane = lax.iota(jnp.int32, 16)
    handles = []
    for j in range(_TPW):
        t = t0 + j
        bb = bufs[j % 2]
        if j >= 2:
            handles[j - 2].wait()
        idx0 = (255 - t) + lane

        @plsc.parallel_loop(0, _PDIM, unroll=2)
        def d_body(d, bb=bb, idx0=idx0):
            fidx = d * _REV_W + idx0
            for c in range(_SLEN // 16):
                v = plsc.load_gather(ph, [fidx + 16 * c])
                bb[d, pl.ds(16 * c, 16)] = v
        handles.append(pltpu.async_copy(bb, outk_hbm.at[t], sem))
    for h in handles[-2:]:
        h.wait()


def _sc_rel_emb(rk):
    mesh = plsc.VectorSubcoreMesh(
        core_axis_name="c", subcore_axis_name="s",
        num_cores=_NC, num_subcores=_NS)
    fn = functools.partial(
        pl.kernel,
        out_type=jax.ShapeDtypeStruct((_SLEN, _PDIM, _SLEN), jnp.float32),
        mesh=mesh,
        scratch_types=[
            pltpu.VMEM((_PDIM * _REV_W,), jnp.float32),
            pltpu.VMEM((_PDIM, _SLEN), jnp.float32),
            pltpu.VMEM((_PDIM, _SLEN), jnp.float32),
            pltpu.SemaphoreType.DMA,
        ],
        compiler_params=pltpu.CompilerParams(needs_layout_passes=False),
    )(_rel_body)
    return fn(rk)


def _relv_body(rv_ref, o_ref):
    g = pl.program_id(0)
    for r in range(_SLEN // 32):
        t = g * (_SLEN // 32) + r
        rolled = pltpu.roll(rv_ref[...], t + 257, 1)
        o_ref[r] = rolled[:, :_SLEN]


def _tc_relv(rv):
    blk = _SLEN // 32
    return pl.pallas_call(
        _relv_body,
        grid=(32,),
        out_shape=jax.ShapeDtypeStruct((_SLEN, _PDIM, _SLEN), jnp.float32),
        in_specs=[pl.BlockSpec((_PDIM, _REV_W), lambda g: (0, 0))],
        out_specs=pl.BlockSpec((blk, _PDIM, _SLEN), lambda g: (g, 0, 0)),
    )(rv)


def kernel(x, pe_k, pe_v, comp, bases, root, bias):
    rk, rv = _tc_rev(pe_k, pe_v)
    yk = _sc_rel_emb(rk)
    out = _tc_out(x, comp, bases, root, bias)
    yv = _tc_relv(rv)
    rel_emb_k = jnp.transpose(yk, (0, 2, 1))
    rel_emb_v = jnp.transpose(yv, (0, 2, 1))
    return out, rel_emb_k, rel_emb_v


# speedup vs baseline: 1.0390x; 1.0390x over previous
import functools

import jax
import jax.numpy as jnp
from jax import lax
from jax.experimental import pallas as pl
from jax.experimental.pallas import tpu as pltpu
from jax.experimental.pallas import tpu_sc as plsc

_SLEN = 256
_DIM = 512
_PDIM = 64
_REV_W = 512

_NC = 1
_NS = 16
_TPW = 16


def _rev_body(pe_k_ref, pe_v_ref, rk_ref, rv_ref):
    u = lax.broadcasted_iota(jnp.int32, (_REV_W, _SLEN + 1), 0)
    r = lax.broadcasted_iota(jnp.int32, (_REV_W, _SLEN + 1), 1)
    sel = (r == jnp.maximum(_SLEN - u, 0)).astype(jnp.float32)
    dn = (((0,), (1,)), ((), ()))
    rk_ref[...] = lax.dot_general(pe_k_ref[pl.ds(0, _SLEN + 1)], sel, dn,
                                  preferred_element_type=jnp.float32)
    rv_ref[...] = lax.dot_general(pe_v_ref[pl.ds(0, _SLEN + 1)], sel, dn,
                                  preferred_element_type=jnp.float32)


def _tc_rev(pe_k, pe_v):
    return pl.pallas_call(
        _rev_body,
        out_shape=(
            jax.ShapeDtypeStruct((_PDIM, _REV_W), jnp.float32),
            jax.ShapeDtypeStruct((_PDIM, _REV_W), jnp.float32),
        ),
        in_specs=[
            pl.BlockSpec(memory_space=pltpu.VMEM),
            pl.BlockSpec(memory_space=pltpu.VMEM),
        ],
        out_specs=(
            pl.BlockSpec(memory_space=pltpu.VMEM),
            pl.BlockSpec(memory_space=pltpu.VMEM),
        ),
    )(pe_k, pe_v)


def _out_body(x_ref, comp_ref, bases_ref, root_ref, bias_ref, o_ref):
    x = x_ref[...]
    w0 = root_ref[...]
    w1 = jnp.zeros((_DIM, _DIM), jnp.float32)
    for b in range(4):
        w0 = w0 + comp_ref[0, b] * bases_ref[b]
        w1 = w1 + comp_ref[1, b] * bases_ref[b]
    row = lax.broadcasted_iota(jnp.int32, (_SLEN, _SLEN), 0)
    col = lax.broadcasted_iota(jnp.int32, (_SLEN, _SLEN), 1)
    tri = (col < row).astype(jnp.float32)
    s = jnp.dot(tri, x, preferred_element_type=jnp.float32)
    n = lax.broadcasted_iota(jnp.int32, (_SLEN, 1), 0).astype(jnp.float32)
    p = s / jnp.maximum(n, 1.0)
    o_ref[...] = (
        jnp.dot(x, w0, preferred_element_type=jnp.float32)
        + jnp.dot(p, w1, preferred_element_type=jnp.float32)
        + bias_ref[...]
    )


def _tc_out(x, comp, bases, root, bias):
    return pl.pallas_call(
        _out_body,
        out_shape=jax.ShapeDtypeStruct((_SLEN, _DIM), jnp.float32),
        in_specs=[
            pl.BlockSpec(memory_space=pltpu.VMEM),
            pl.BlockSpec(memory_space=pltpu.SMEM),
            pl.BlockSpec(memory_space=pltpu.VMEM),
            pl.BlockSpec(memory_space=pltpu.VMEM),
            pl.BlockSpec(memory_space=pltpu.VMEM),
        ],
        out_specs=pl.BlockSpec(memory_space=pltpu.VMEM),
    )(x, comp, bases, root, bias.reshape(1, _DIM))


def _rel_body(rk_hbm, outk_hbm, ph, bb0, bb1, sem):
    wid = lax.axis_index("s") * _NC + lax.axis_index("c")
    t0 = wid * _TPW
    bufs = (bb0, bb1)

    hs = [pltpu.async_copy(rk_hbm.at[d], ph.at[pl.ds(d * _REV_W, _REV_W)],
                           sem) for d in range(_PDIM)]
    for h in hs:
        h.wait()

    lane = lax.iota(jnp.int32, 16)
    handles = []
    for j in range(_TPW):
        t = t0 + j
        bb = bufs[j % 2]
        if j >= 2:
            handles[j - 2].wait()
        idx0 = (255 - t) + lane

        @plsc.parallel_loop(0, _PDIM, unroll=2)
        def d_body(d, bb=bb, idx0=idx0):
            fidx = d * _REV_W + idx0
            for c in range(_SLEN // 16):
                v = plsc.load_gather(ph, [fidx + 16 * c])
                bb[d, pl.ds(16 * c, 16)] = v
        handles.append(pltpu.async_copy(bb, outk_hbm.at[t], sem))
    for h in handles[-2:]:
        h.wait()


def _sc_rel_emb(rk):
    mesh = plsc.VectorSubcoreMesh(
        core_axis_name="c", subcore_axis_name="s",
        num_cores=_NC, num_subcores=_NS)
    fn = functools.partial(
        pl.kernel,
        out_type=jax.ShapeDtypeStruct((_SLEN, _PDIM, _SLEN), jnp.float32),
        mesh=mesh,
        scratch_types=[
            pltpu.VMEM((_PDIM * _REV_W,), jnp.float32),
            pltpu.VMEM((_PDIM, _SLEN), jnp.float32),
            pltpu.VMEM((_PDIM, _SLEN), jnp.float32),
            pltpu.SemaphoreType.DMA,
        ],
        compiler_params=pltpu.CompilerParams(needs_layout_passes=False),
    )(_rel_body)
    return fn(rk)


def _relv_body(rv_ref, o_ref):
    g = pl.program_id(0)
    for r in range(_SLEN // 32):
        t = g * (_SLEN // 32) + r
        rolled = pltpu.roll(rv_ref[...], t + 257, 1)
        o_ref[r] = rolled[:, :_SLEN]


def _tc_relv(rv):
    blk = _SLEN // 32
    return pl.pallas_call(
        _relv_body,
        grid=(32,),
        out_shape=jax.ShapeDtypeStruct((_SLEN, _PDIM, _SLEN), jnp.float32),
        in_specs=[pl.BlockSpec((_PDIM, _REV_W), lambda g: (0, 0))],
        out_specs=pl.BlockSpec((blk, _PDIM, _SLEN), lambda g: (g, 0, 0)),
    )(rv)


def kernel(x, pe_k, pe_v, comp, bases, root, bias):
    rk, rv = _tc_rev(pe_k, pe_v)
    yk = _sc_rel_emb(rk)
    out = _tc_out(x, comp, bases, root, bias)
    yv = _tc_relv(rv)
    rel_emb_k = jnp.transpose(yk, (0, 2, 1))
    rel_emb_v = jnp.transpose(yv, (0, 2, 1))
    return out, rel_emb_k, rel_emb_v
